# baseline (speedup 1.0000x reference)
import functools

import jax
import jax.numpy as jnp
from jax import lax
from jax.experimental import pallas as pl
from jax.experimental.pallas import tpu as pltpu
from jax.experimental.pallas import tpu_sc as plsc

D_IN = 768
L = 32768
N = 4096
K = 32

BM = 256
BN = 4096
G = 32
NGRP = L // G
NB = 512
NCAND = K * G

_NC = 2
_NS = 16
_NW = _NC * _NS

_SPLIT = 4


def _encode_body(x_ref, wenc_ref, benc_ref, bdec_ref, acts_ref, gmax_ref):
    xc = (x_ref[...] - bdec_ref[...]).astype(jnp.bfloat16)
    pre = lax.dot_general(
        xc, wenc_ref[...],
        dimension_numbers=(((1,), (1,)), ((), ())),
        preferred_element_type=jnp.float32,
    ) + benc_ref[...]
    a = jnp.maximum(pre, 0.0)
    acts_ref[...] = a
    gmax_ref[...] = jnp.max(a.reshape(BM, BN // G, G), axis=2)


def _encode(x, W_enc_bf, b_enc, b_dec):
    n = x.shape[0]
    return pl.pallas_call(
        _encode_body,
        grid=(L // BN, n // BM),
        in_specs=[
            pl.BlockSpec((BM, D_IN), lambda j, i: (i, 0)),
            pl.BlockSpec((BN, D_IN), lambda j, i: (j, 0)),
            pl.BlockSpec((1, BN), lambda j, i: (0, j)),
            pl.BlockSpec((1, D_IN), lambda j, i: (0, 0)),
        ],
        out_specs=[
            pl.BlockSpec((BM, BN), lambda j, i: (i, j)),
            pl.BlockSpec((BM, BN // G), lambda j, i: (i, j)),
        ],
        out_shape=[
            jax.ShapeDtypeStruct((n, L), jnp.float32),
            jax.ShapeDtypeStruct((n, NGRP), jnp.float32),
        ],
    )(x, W_enc_bf, b_enc.reshape(1, L), b_dec.reshape(1, D_IN))


def _b1_body(gmax_ref, grow_ref, cidx_ref):
    i = pl.program_id(0)
    vals = gmax_ref[...]
    ga = lax.broadcasted_iota(jnp.int32, (NB, NGRP), 1)
    tok = i * NB + lax.broadcasted_iota(jnp.int32, (NB, 1), 0)
    offs = lax.broadcasted_iota(jnp.int32, (NB, G), 1)
    rows, cols = [], []
    for _t in range(K):
        m = jnp.max(vals, axis=1, keepdims=True)
        sel = jnp.min(jnp.where(vals == m, ga, NGRP), axis=1, keepdims=True)
        rows.append(tok * NGRP + sel)
        cols.append(sel * G + offs)
        vals = jnp.where(ga == sel, -1.0, vals)
    grow_ref[0] = jnp.concatenate(rows, axis=1)
    cidx_ref[...] = jnp.concatenate(cols, axis=1)


def _b1(gmax):
    n = gmax.shape[0]
    return pl.pallas_call(
        _b1_body,
        grid=(n // NB,),
        in_specs=[pl.BlockSpec((NB, NGRP), lambda i: (i, 0))],
        out_specs=[
            pl.BlockSpec((1, NB, K), lambda i: (i, 0, 0)),
            pl.BlockSpec((NB, NCAND), lambda i: (i, 0)),
        ],
        out_shape=[
            jax.ShapeDtypeStruct((n // NB, NB, K), jnp.int32),
            jax.ShapeDtypeStruct((n, NCAND), jnp.int32),
        ],
    )(gmax)


_CH = 128


def _c_gather(grow_flat, acts_rows):
    nrows = grow_flat.shape[0]
    rpw = nrows // _NW
    half = rpw // 2

    def body(grow_hbm, acts_hbm, cand_hbm, idx_v, rows_v, sem):
        wid = lax.axis_index("s") * _NC + lax.axis_index("c")
        base = wid * rpw
        pltpu.sync_copy(grow_hbm.at[pl.ds(base, rpw)], idx_v)
        for h in range(2):
            cps = []
            for c in range(half // _CH):
                cps.append(pltpu.async_copy(
                    acts_hbm.at[idx_v.at[pl.ds(h * half + c * _CH, _CH)]],
                    rows_v.at[pl.ds(c * _CH, _CH)], sem))
            for cp in cps:
                cp.wait()
            pltpu.sync_copy(rows_v, cand_hbm.at[pl.ds(base + h * half, half)])

    f = functools.partial(
        pl.kernel,
        out_type=jax.ShapeDtypeStruct((nrows, G), jnp.float32),
        mesh=plsc.VectorSubcoreMesh(core_axis_name="c", subcore_axis_name="s"),
        compiler_params=pltpu.CompilerParams(use_tc_tiling_on_sc=False),
        scratch_types=[
            pltpu.VMEM((rpw,), jnp.int32),
            pltpu.VMEM((half, G), jnp.float32),
            pltpu.SemaphoreType.DMA,
        ],
    )(body)
    return f(grow_flat, acts_rows)


def _b2_body(cand_ref, cidx_ref, ta_ref, ti_ref):
    vals = cand_ref[...]
    ci = cidx_ref[...]
    tas, tis = [], []
    for _t in range(K):
        m = jnp.max(vals, axis=1, keepdims=True)
        sel = jnp.min(jnp.where(vals == m, ci, jnp.int32(1 << 30)),
                      axis=1, keepdims=True)
        tas.append(m)
        tis.append(sel)
        vals = jnp.where(ci == sel, -1.0, vals)
    ta_ref[0] = jnp.concatenate(tas, axis=1)
    ti_ref[0] = jnp.concatenate(tis, axis=1)


def _b2(cand, cidx):
    n = cand.shape[0]
    return pl.pallas_call(
        _b2_body,
        grid=(n // NB,),
        in_specs=[
            pl.BlockSpec((NB, NCAND), lambda i: (i, 0)),
            pl.BlockSpec((NB, NCAND), lambda i: (i, 0)),
        ],
        out_specs=[
            pl.BlockSpec((1, NB, K), lambda i: (i, 0, 0)),
            pl.BlockSpec((1, NB, K), lambda i: (i, 0, 0)),
        ],
        out_shape=[
            jax.ShapeDtypeStruct((n // NB, NB, K), jnp.float32),
            jax.ShapeDtypeStruct((n // NB, NB, K), jnp.int32),
        ],
    )(cand, cidx)


_XCH = 16
_DH = D_IN // 2
_NH = _DH // 16


def _d_decode(W_dec, tidx_flat, ta_flat, x_flat, b_dec):
    n = x_flat.shape[0] // D_IN
    tpw = n // _NW

    def body(wdec_hbm, tidx_hbm, ta_hbm, x_hbm, bdec_hbm,
             sae_hbm, esum_hbm, colsum_hbm, sumsq_hbm,
             tidx_v, ta_v, bdec_v, rows0_v, rows1_v, x_v, sae_v,
             colsum_v, esum_v, sumsq_v, sem0, sem1):
        wid = lax.axis_index("s") * _NC + lax.axis_index("c")
        tbase = wid * tpw
        pltpu.sync_copy(tidx_hbm.at[pl.ds(tbase * K, tpw * K)], tidx_v)
        pltpu.sync_copy(ta_hbm.at[pl.ds(tbase * K, tpw * K)], ta_v)
        pltpu.sync_copy(bdec_hbm, bdec_v)
        zero16 = jnp.zeros((16,), jnp.float32)
        esum_v[...] = zero16
        sumsq_v[...] = zero16

        def zcol(i, c):
            colsum_v[pl.ds(i * 16, 16)] = zero16
            return c
        lax.fori_loop(0, D_IN // 16, zcol, 0)

        def gather(tl, rows_ref, sem):
            return pltpu.async_copy(
                wdec_hbm.at[tidx_v.at[pl.ds(tl * K, K)]], rows_ref, sem)

        def compute(tl, rows_ref):
            tchunk = lax.rem(tl, jnp.int32(_XCH))
            for h in range(2):
                def kbody(k, carry):
                    a16 = plsc.load_gather(
                        ta_v, [jnp.full((16,), 0, jnp.int32) + (tl * K + k)])
                    return tuple(
                        carry[c]
                        + a16 * rows_ref[k, pl.ds(h * _DH + c * 16, 16)]
                        for c in range(_NH))
                init = tuple(bdec_v[pl.ds(h * _DH + c * 16, 16)]
                             for c in range(_NH))
                acc = lax.fori_loop(0, K, kbody, init)
                for c in range(_NH):
                    d0 = h * _DH + c * 16
                    xw = x_v[pl.ds(tchunk * D_IN + d0, 16)]
                    sae_v[pl.ds(tchunk * D_IN + d0, 16)] = acc[c]
                    e = acc[c] - xw
                    esum_v[...] = esum_v[...] + e * e
                    sumsq_v[...] = sumsq_v[...] + xw * xw
                    colsum_v[pl.ds(d0, 16)] = colsum_v[pl.ds(d0, 16)] + xw

        def pair_body(p, c):
            @pl.when(lax.rem(p, jnp.int32(_XCH // 2)) == 0)
            def _():
                pltpu.sync_copy(
                    x_hbm.at[pl.ds((tbase + p * 2) * D_IN, _XCH * D_IN)],
                    x_v)
            h0 = gather(p * 2, rows0_v, sem0)
            h1 = gather(p * 2 + 1, rows1_v, sem1)
            h0.wait()
            compute(p * 2, rows0_v)
            h1.wait()
            compute(p * 2 + 1, rows1_v)

            @pl.when(lax.rem(p, jnp.int32(_XCH // 2)) == _XCH // 2 - 1)
            def _():
                pltpu.sync_copy(
                    sae_v,
                    sae_hbm.at[pl.ds((tbase + (p * 2 - _XCH + 2)) * D_IN,
                                     _XCH * D_IN)])
            return c
        lax.fori_loop(0, tpw // 2, pair_body, 0)

        pltpu.sync_copy(esum_v, esum_hbm.at[wid])
        pltpu.sync_copy(sumsq_v, sumsq_hbm.at[wid])
        pltpu.sync_copy(colsum_v, colsum_hbm.at[wid])

    f = functools.partial(
        pl.kernel,
        out_type=[
            jax.ShapeDtypeStruct((n * D_IN,), jnp.float32),
            jax.ShapeDtypeStruct((_NW, 16), jnp.float32),
            jax.ShapeDtypeStruct((_NW, D_IN), jnp.float32),
            jax.ShapeDtypeStruct((_NW, 16), jnp.float32),
        ],
        mesh=plsc.VectorSubcoreMesh(core_axis_name="c", subcore_axis_name="s"),
        compiler_params=pltpu.CompilerParams(
            use_tc_tiling_on_sc=False, needs_layout_passes=False),
        scratch_types=[
            pltpu.VMEM((tpw * K,), jnp.int32),
            pltpu.VMEM((tpw * K,), jnp.float32),
            pltpu.VMEM((D_IN,), jnp.float32),
            pltpu.VMEM((K, D_IN), jnp.float32),
            pltpu.VMEM((K, D_IN), jnp.float32),
            pltpu.VMEM((_XCH * D_IN,), jnp.float32),
            pltpu.VMEM((_XCH * D_IN,), jnp.float32),
            pltpu.VMEM((D_IN,), jnp.float32),
            pltpu.VMEM((16,), jnp.float32),
            pltpu.VMEM((16,), jnp.float32),
            pltpu.SemaphoreType.DMA,
            pltpu.SemaphoreType.DMA,
        ],
    )(body)
    return f(W_dec, tidx_flat, ta_flat, x_flat, b_dec)


def kernel(x, W_enc, b_enc, W_dec, b_dec):
    W_enc_bf = W_enc.astype(jnp.bfloat16)
    nh = N // _SPLIT
    parts = []
    for s in range(_SPLIT):
        xs = lax.slice_in_dim(x, s * nh, (s + 1) * nh, axis=0)
        acts, gmax = _encode(xs, W_enc_bf, b_enc, b_dec)
        grow3, cidx = _b1(gmax)
        cand_rows = _c_gather(grow3.reshape(nh * K),
                              acts.reshape(nh * NGRP, G))
        ta3, ti3 = _b2(cand_rows.reshape(nh, NCAND), cidx)
        sae_flat, esum, colsum, sumsq = _d_decode(
            W_dec, ti3.reshape(nh * K), ta3.reshape(nh * K),
            xs.reshape(nh * D_IN), b_dec)
        parts.append((sae_flat.reshape(nh, D_IN), ta3.reshape(nh, K),
                      ti3.reshape(nh, K), esum, colsum, sumsq))

    sae_out = jnp.concatenate([p[0] for p in parts], axis=0)
    top_acts = jnp.concatenate([p[1] for p in parts], axis=0)
    top_indices = jnp.concatenate([p[2] for p in parts], axis=0)
    esum_t = sum(jnp.sum(p[3]) for p in parts)
    colsum_t = sum(jnp.sum(p[4], axis=0) for p in parts)
    sumsq_t = sum(jnp.sum(p[5]) for p in parts)
    total_variance = sumsq_t - jnp.sum(colsum_t * colsum_t) / N
    fvu = esum_t / total_variance
    auxk_loss = jnp.zeros((), dtype=x.dtype)
    return sae_out, top_acts, top_indices, fvu, auxk_loss


# speedup vs baseline: 1.0161x; 1.0161x over previous
import functools

import jax
import jax.numpy as jnp
from jax import lax
from jax.experimental import pallas as pl
from jax.experimental.pallas import tpu as pltpu
from jax.experimental.pallas import tpu_sc as plsc

D_IN = 768
L = 32768
N = 4096
K = 32

BM = 256
BN = 4096
G = 32
NGRP = L // G
NB = 512
NCAND = K * G

_NC = 2
_NS = 16
_NW = _NC * _NS

_SPLIT = 2


def _encode_body(x_ref, wenc_ref, benc_ref, bdec_ref, acts_ref, gmax_ref):
    xc = (x_ref[...] - bdec_ref[...]).astype(jnp.bfloat16)
    pre = lax.dot_general(
        xc, wenc_ref[...],
        dimension_numbers=(((1,), (1,)), ((), ())),
        preferred_element_type=jnp.float32,
    ) + benc_ref[...]
    a = jnp.maximum(pre, 0.0)
    acts_ref[...] = a
    gmax_ref[...] = jnp.max(a.reshape(BM, BN // G, G), axis=2)


def _encode(x, W_enc_bf, b_enc, b_dec):
    n = x.shape[0]
    return pl.pallas_call(
        _encode_body,
        grid=(L // BN, n // BM),
        in_specs=[
            pl.BlockSpec((BM, D_IN), lambda j, i: (i, 0)),
            pl.BlockSpec((BN, D_IN), lambda j, i: (j, 0)),
            pl.BlockSpec((1, BN), lambda j, i: (0, j)),
            pl.BlockSpec((1, D_IN), lambda j, i: (0, 0)),
        ],
        out_specs=[
            pl.BlockSpec((BM, BN), lambda j, i: (i, j)),
            pl.BlockSpec((BM, BN // G), lambda j, i: (i, j)),
        ],
        out_shape=[
            jax.ShapeDtypeStruct((n, L), jnp.float32),
            jax.ShapeDtypeStruct((n, NGRP), jnp.float32),
        ],
    )(x, W_enc_bf, b_enc.reshape(1, L), b_dec.reshape(1, D_IN))


def _b1_body(gmax_ref, grow_ref, cidx_ref):
    i = pl.program_id(0)
    vals = gmax_ref[...]
    ga = lax.broadcasted_iota(jnp.int32, (NB, NGRP), 1)
    tok = i * NB + lax.broadcasted_iota(jnp.int32, (NB, 1), 0)
    offs = lax.broadcasted_iota(jnp.int32, (NB, G), 1)
    rows, cols = [], []
    for _t in range(K):
        sel = jnp.argmax(vals, axis=1).astype(jnp.int32).reshape(NB, 1)
        rows.append(tok * NGRP + sel)
        cols.append(sel * G + offs)
        vals = jnp.where(ga == sel, -1.0, vals)
    grow_ref[0] = jnp.concatenate(rows, axis=1)
    cidx_ref[...] = jnp.concatenate(cols, axis=1)


def _b1(gmax):
    n = gmax.shape[0]
    return pl.pallas_call(
        _b1_body,
        grid=(n // NB,),
        in_specs=[pl.BlockSpec((NB, NGRP), lambda i: (i, 0))],
        out_specs=[
            pl.BlockSpec((1, NB, K), lambda i: (i, 0, 0)),
            pl.BlockSpec((NB, NCAND), lambda i: (i, 0)),
        ],
        out_shape=[
            jax.ShapeDtypeStruct((n // NB, NB, K), jnp.int32),
            jax.ShapeDtypeStruct((n, NCAND), jnp.int32),
        ],
    )(gmax)


_CH = 128


def _c_gather(grow_flat, acts_rows):
    nrows = grow_flat.shape[0]
    rpw = nrows // _NW
    half = rpw // 2

    def body(grow_hbm, acts_hbm, cand_hbm, idx_v, rows_v, sem):
        wid = lax.axis_index("s") * _NC + lax.axis_index("c")
        base = wid * rpw
        pltpu.sync_copy(grow_hbm.at[pl.ds(base, rpw)], idx_v)
        for h in range(2):
            cps = []
            for c in range(half // _CH):
                cps.append(pltpu.async_copy(
                    acts_hbm.at[idx_v.at[pl.ds(h * half + c * _CH, _CH)]],
                    rows_v.at[pl.ds(c * _CH, _CH)], sem))
            for cp in cps:
                cp.wait()
            pltpu.sync_copy(rows_v, cand_hbm.at[pl.ds(base + h * half, half)])

    f = functools.partial(
        pl.kernel,
        out_type=jax.ShapeDtypeStruct((nrows, G), jnp.float32),
        mesh=plsc.VectorSubcoreMesh(core_axis_name="c", subcore_axis_name="s"),
        compiler_params=pltpu.CompilerParams(use_tc_tiling_on_sc=False),
        scratch_types=[
            pltpu.VMEM((rpw,), jnp.int32),
            pltpu.VMEM((half, G), jnp.float32),
            pltpu.SemaphoreType.DMA,
        ],
    )(body)
    return f(grow_flat, acts_rows)


def _b2_body(cand_ref, cidx_ref, ta_ref, ti_ref):
    vals = cand_ref[...]
    ci = cidx_ref[...]
    tas, tis = [], []
    for _t in range(K):
        m = jnp.max(vals, axis=1, keepdims=True)
        sel = jnp.min(jnp.where(vals == m, ci, jnp.int32(1 << 30)),
                      axis=1, keepdims=True)
        tas.append(m)
        tis.append(sel)
        vals = jnp.where(ci == sel, -1.0, vals)
    ta_ref[0] = jnp.concatenate(tas, axis=1)
    ti_ref[0] = jnp.concatenate(tis, axis=1)


def _b2(cand, cidx):
    n = cand.shape[0]
    return pl.pallas_call(
        _b2_body,
        grid=(n // NB,),
        in_specs=[
            pl.BlockSpec((NB, NCAND), lambda i: (i, 0)),
            pl.BlockSpec((NB, NCAND), lambda i: (i, 0)),
        ],
        out_specs=[
            pl.BlockSpec((1, NB, K), lambda i: (i, 0, 0)),
            pl.BlockSpec((1, NB, K), lambda i: (i, 0, 0)),
        ],
        out_shape=[
            jax.ShapeDtypeStruct((n // NB, NB, K), jnp.float32),
            jax.ShapeDtypeStruct((n // NB, NB, K), jnp.int32),
        ],
    )(cand, cidx)


_XCH = 16
_DH = D_IN // 2
_NH = _DH // 16


def _d_decode(W_dec, tidx_flat, ta_flat, x_flat, b_dec):
    n = x_flat.shape[0] // D_IN
    tpw = n // _NW

    def body(wdec_hbm, tidx_hbm, ta_hbm, x_hbm, bdec_hbm,
             sae_hbm, esum_hbm, colsum_hbm, sumsq_hbm,
             tidx_v, ta_v, bdec_v, rows0_v, rows1_v, x_v, sae_v,
             colsum_v, esum_v, sumsq_v, sem0, sem1):
        wid = lax.axis_index("s") * _NC + lax.axis_index("c")
        tbase = wid * tpw
        pltpu.sync_copy(tidx_hbm.at[pl.ds(tbase * K, tpw * K)], tidx_v)
        pltpu.sync_copy(ta_hbm.at[pl.ds(tbase * K, tpw * K)], ta_v)
        pltpu.sync_copy(bdec_hbm, bdec_v)
        zero16 = jnp.zeros((16,), jnp.float32)
        esum_v[...] = zero16
        sumsq_v[...] = zero16

        def zcol(i, c):
            colsum_v[pl.ds(i * 16, 16)] = zero16
            return c
        lax.fori_loop(0, D_IN // 16, zcol, 0)

        def gather(tl, rows_ref, sem):
            return pltpu.async_copy(
                wdec_hbm.at[tidx_v.at[pl.ds(tl * K, K)]], rows_ref, sem)

        def compute(tl, rows_ref):
            tchunk = lax.rem(tl, jnp.int32(_XCH))
            for h in range(2):
                def kbody(k, carry):
                    a16 = plsc.load_gather(
                        ta_v, [jnp.full((16,), 0, jnp.int32) + (tl * K + k)])
                    return tuple(
                        carry[c]
                        + a16 * rows_ref[k, pl.ds(h * _DH + c * 16, 16)]
                        for c in range(_NH))
                init = tuple(bdec_v[pl.ds(h * _DH + c * 16, 16)]
                             for c in range(_NH))
                acc = lax.fori_loop(0, K, kbody, init)
                for c in range(_NH):
                    d0 = h * _DH + c * 16
                    xw = x_v[pl.ds(tchunk * D_IN + d0, 16)]
                    sae_v[pl.ds(tchunk * D_IN + d0, 16)] = acc[c]
                    e = acc[c] - xw
                    esum_v[...] = esum_v[...] + e * e
                    sumsq_v[...] = sumsq_v[...] + xw * xw
                    colsum_v[pl.ds(d0, 16)] = colsum_v[pl.ds(d0, 16)] + xw

        def pair_body(p, c):
            @pl.when(lax.rem(p, jnp.int32(_XCH // 2)) == 0)
            def _():
                pltpu.sync_copy(
                    x_hbm.at[pl.ds((tbase + p * 2) * D_IN, _XCH * D_IN)],
                    x_v)
            h0 = gather(p * 2, rows0_v, sem0)
            h1 = gather(p * 2 + 1, rows1_v, sem1)
            h0.wait()
            compute(p * 2, rows0_v)
            h1.wait()
            compute(p * 2 + 1, rows1_v)

            @pl.when(lax.rem(p, jnp.int32(_XCH // 2)) == _XCH // 2 - 1)
            def _():
                pltpu.sync_copy(
                    sae_v,
                    sae_hbm.at[pl.ds((tbase + (p * 2 - _XCH + 2)) * D_IN,
                                     _XCH * D_IN)])
            return c
        lax.fori_loop(0, tpw // 2, pair_body, 0)

        pltpu.sync_copy(esum_v, esum_hbm.at[wid])
        pltpu.sync_copy(sumsq_v, sumsq_hbm.at[wid])
        pltpu.sync_copy(colsum_v, colsum_hbm.at[wid])

    f = functools.partial(
        pl.kernel,
        out_type=[
            jax.ShapeDtypeStruct((n * D_IN,), jnp.float32),
            jax.ShapeDtypeStruct((_NW, 16), jnp.float32),
            jax.ShapeDtypeStruct((_NW, D_IN), jnp.float32),
            jax.ShapeDtypeStruct((_NW, 16), jnp.float32),
        ],
        mesh=plsc.VectorSubcoreMesh(core_axis_name="c", subcore_axis_name="s"),
        compiler_params=pltpu.CompilerParams(
            use_tc_tiling_on_sc=False, needs_layout_passes=False),
        scratch_types=[
            pltpu.VMEM((tpw * K,), jnp.int32),
            pltpu.VMEM((tpw * K,), jnp.float32),
            pltpu.VMEM((D_IN,), jnp.float32),
            pltpu.VMEM((K, D_IN), jnp.float32),
            pltpu.VMEM((K, D_IN), jnp.float32),
            pltpu.VMEM((_XCH * D_IN,), jnp.float32),
            pltpu.VMEM((_XCH * D_IN,), jnp.float32),
            pltpu.VMEM((D_IN,), jnp.float32),
            pltpu.VMEM((16,), jnp.float32),
            pltpu.VMEM((16,), jnp.float32),
            pltpu.SemaphoreType.DMA,
            pltpu.SemaphoreType.DMA,
        ],
    )(body)
    return f(W_dec, tidx_flat, ta_flat, x_flat, b_dec)


def kernel(x, W_enc, b_enc, W_dec, b_dec):
    W_enc_bf = W_enc.astype(jnp.bfloat16)
    nh = N // _SPLIT
    parts = []
    for s in range(_SPLIT):
        xs = lax.slice_in_dim(x, s * nh, (s + 1) * nh, axis=0)
        acts, gmax = _encode(xs, W_enc_bf, b_enc, b_dec)
        grow3, cidx = _b1(gmax)
        cand_rows = _c_gather(grow3.reshape(nh * K),
                              acts.reshape(nh * NGRP, G))
        ta3, ti3 = _b2(cand_rows.reshape(nh, NCAND), cidx)
        sae_flat, esum, colsum, sumsq = _d_decode(
            W_dec, ti3.reshape(nh * K), ta3.reshape(nh * K),
            xs.reshape(nh * D_IN), b_dec)
        parts.append((sae_flat.reshape(nh, D_IN), ta3.reshape(nh, K),
                      ti3.reshape(nh, K), esum, colsum, sumsq))

    sae_out = jnp.concatenate([p[0] for p in parts], axis=0)
    top_acts = jnp.concatenate([p[1] for p in parts], axis=0)
    top_indices = jnp.concatenate([p[2] for p in parts], axis=0)
    esum_t = sum(jnp.sum(p[3]) for p in parts)
    colsum_t = sum(jnp.sum(p[4], axis=0) for p in parts)
    sumsq_t = sum(jnp.sum(p[5]) for p in parts)
    total_variance = sumsq_t - jnp.sum(colsum_t * colsum_t) / N
    fvu = esum_t / total_variance
    auxk_loss = jnp.zeros((), dtype=x.dtype)
    return sae_out, top_acts, top_indices, fvu, auxk_loss


# speedup vs baseline: 1.0892x; 1.0719x over previous
import functools

import jax
import jax.numpy as jnp
from jax import lax
from jax.experimental import pallas as pl
from jax.experimental.pallas import tpu as pltpu
from jax.experimental.pallas import tpu_sc as plsc

D_IN = 768
L = 32768
N = 4096
K = 32

BM = 256
BN = 4096
G = 32
NGRP = L // G
NB = 512
NCAND = K * G

_NC = 2
_NS = 16
_NW = _NC * _NS

_SPLIT = 2


def _encode_body(x_ref, wenc_ref, benc_ref, bdec_ref, acts_ref, gmax_ref):
    xc = (x_ref[...] - bdec_ref[...]).astype(jnp.bfloat16)
    pre = lax.dot_general(
        xc, wenc_ref[...],
        dimension_numbers=(((1,), (1,)), ((), ())),
        preferred_element_type=jnp.float32,
    ) + benc_ref[...]
    a = jnp.maximum(pre, 0.0)
    acts_ref[...] = a
    gmax_ref[...] = jnp.max(a.reshape(BM, BN // G, G), axis=2)


def _encode(x, W_enc_bf, b_enc, b_dec):
    n = x.shape[0]
    return pl.pallas_call(
        _encode_body,
        grid=(L // BN, n // BM),
        in_specs=[
            pl.BlockSpec((BM, D_IN), lambda j, i: (i, 0)),
            pl.BlockSpec((BN, D_IN), lambda j, i: (j, 0)),
            pl.BlockSpec((1, BN), lambda j, i: (0, j)),
            pl.BlockSpec((1, D_IN), lambda j, i: (0, 0)),
        ],
        out_specs=[
            pl.BlockSpec((BM, BN), lambda j, i: (i, j)),
            pl.BlockSpec((BM, BN // G), lambda j, i: (i, j)),
        ],
        out_shape=[
            jax.ShapeDtypeStruct((n, L), jnp.float32),
            jax.ShapeDtypeStruct((n, NGRP), jnp.float32),
        ],
    )(x, W_enc_bf, b_enc.reshape(1, L), b_dec.reshape(1, D_IN))


def _b1_body(gmax_ref, grow_ref, cidx_ref):
    i = pl.program_id(0)
    vals = gmax_ref[...]
    ga = lax.broadcasted_iota(jnp.int32, (NB, NGRP), 1)
    tok = i * NB + lax.broadcasted_iota(jnp.int32, (NB, 1), 0)
    offs = lax.broadcasted_iota(jnp.int32, (NB, G), 1)
    rows, cols = [], []
    for _t in range(K):
        sel = jnp.argmax(vals, axis=1).astype(jnp.int32).reshape(NB, 1)
        rows.append(tok * NGRP + sel)
        cols.append(sel * G + offs)
        vals = jnp.where(ga == sel, -1.0, vals)
    grow_ref[0] = jnp.concatenate(rows, axis=1)
    cidx_ref[...] = jnp.concatenate(cols, axis=1)


def _b1(gmax):
    n = gmax.shape[0]
    return pl.pallas_call(
        _b1_body,
        grid=(n // NB,),
        in_specs=[pl.BlockSpec((NB, NGRP), lambda i: (i, 0))],
        out_specs=[
            pl.BlockSpec((1, NB, K), lambda i: (i, 0, 0)),
            pl.BlockSpec((NB, NCAND), lambda i: (i, 0)),
        ],
        out_shape=[
            jax.ShapeDtypeStruct((n // NB, NB, K), jnp.int32),
            jax.ShapeDtypeStruct((n, NCAND), jnp.int32),
        ],
    )(gmax)


_CH = 128


def _c_gather(grow_flat, acts_rows):
    nrows = grow_flat.shape[0]
    rpw = nrows // _NW
    half = rpw // 2

    def body(grow_hbm, acts_hbm, cand_hbm, idx_v, rows_v, sem):
        wid = lax.axis_index("s") * _NC + lax.axis_index("c")
        base = wid * rpw
        pltpu.sync_copy(grow_hbm.at[pl.ds(base, rpw)], idx_v)
        for h in range(2):
            cps = []
            for c in range(half // _CH):
                cps.append(pltpu.async_copy(
                    acts_hbm.at[idx_v.at[pl.ds(h * half + c * _CH, _CH)]],
                    rows_v.at[pl.ds(c * _CH, _CH)], sem))
            for cp in cps:
                cp.wait()
            pltpu.sync_copy(rows_v, cand_hbm.at[pl.ds(base + h * half, half)])

    f = functools.partial(
        pl.kernel,
        out_type=jax.ShapeDtypeStruct((nrows, G), jnp.float32),
        mesh=plsc.VectorSubcoreMesh(core_axis_name="c", subcore_axis_name="s"),
        compiler_params=pltpu.CompilerParams(use_tc_tiling_on_sc=False),
        scratch_types=[
            pltpu.VMEM((rpw,), jnp.int32),
            pltpu.VMEM((half, G), jnp.float32),
            pltpu.SemaphoreType.DMA,
        ],
    )(body)
    return f(grow_flat, acts_rows)


def _b2_body(cand_ref, cidx_ref, ta_ref, ti_ref):
    vals = cand_ref[...]
    ci = cidx_ref[...]
    tas, tis = [], []
    for _t in range(K):
        m = jnp.max(vals, axis=1, keepdims=True)
        sel = jnp.min(jnp.where(vals == m, ci, jnp.int32(1 << 30)),
                      axis=1, keepdims=True)
        tas.append(m)
        tis.append(sel)
        vals = jnp.where(ci == sel, -1.0, vals)
    ta_ref[0] = jnp.concatenate(tas, axis=1)
    ti_ref[0] = jnp.concatenate(tis, axis=1)


def _b2(cand, cidx):
    n = cand.shape[0]
    return pl.pallas_call(
        _b2_body,
        grid=(n // NB,),
        in_specs=[
            pl.BlockSpec((NB, NCAND), lambda i: (i, 0)),
            pl.BlockSpec((NB, NCAND), lambda i: (i, 0)),
        ],
        out_specs=[
            pl.BlockSpec((1, NB, K), lambda i: (i, 0, 0)),
            pl.BlockSpec((1, NB, K), lambda i: (i, 0, 0)),
        ],
        out_shape=[
            jax.ShapeDtypeStruct((n // NB, NB, K), jnp.float32),
            jax.ShapeDtypeStruct((n // NB, NB, K), jnp.int32),
        ],
    )(cand, cidx)


_XCH = 16
_DH = D_IN // 2
_NH = _DH // 16


def _d_decode(W_dec, tidx_flat, ta_flat, x_flat, b_dec):
    n = x_flat.shape[0] // D_IN
    tpw = n // _NW

    def body(wdec_hbm, tidx_hbm, ta_hbm, x_hbm, bdec_hbm,
             sae_hbm, esum_hbm, colsum_hbm, sumsq_hbm,
             tidx_v, ta_v, bdec_v, rows0_v, rows1_v, x_v, sae_v,
             colsum_v, esum_v, sumsq_v, sem0, sem1):
        wid = lax.axis_index("s") * _NC + lax.axis_index("c")
        tbase = wid * tpw
        pltpu.sync_copy(tidx_hbm.at[pl.ds(tbase * K, tpw * K)], tidx_v)
        pltpu.sync_copy(ta_hbm.at[pl.ds(tbase * K, tpw * K)], ta_v)
        pltpu.sync_copy(bdec_hbm, bdec_v)
        zero16 = jnp.zeros((16,), jnp.float32)
        esum_v[...] = zero16
        sumsq_v[...] = zero16

        def zcol(i, c):
            colsum_v[pl.ds(i * 16, 16)] = zero16
            return c
        lax.fori_loop(0, D_IN // 16, zcol, 0)

        def gather(tl, rows_ref, sem):
            return pltpu.async_copy(
                wdec_hbm.at[tidx_v.at[pl.ds(tl * K, K)]], rows_ref, sem)

        def compute(tl, rows_ref):
            tchunk = lax.rem(tl, jnp.int32(_XCH))
            for h in range(2):
                def kbody(k, carry):
                    a16 = plsc.load_gather(
                        ta_v, [jnp.full((16,), 0, jnp.int32) + (tl * K + k)])
                    return tuple(
                        carry[c]
                        + a16 * rows_ref[k, pl.ds(h * _DH + c * 16, 16)]
                        for c in range(_NH))
                init = tuple(bdec_v[pl.ds(h * _DH + c * 16, 16)]
                             for c in range(_NH))
                acc = lax.fori_loop(0, K, kbody, init)
                for c in range(_NH):
                    d0 = h * _DH + c * 16
                    xw = x_v[pl.ds(tchunk * D_IN + d0, 16)]
                    sae_v[pl.ds(tchunk * D_IN + d0, 16)] = acc[c]
                    e = acc[c] - xw
                    esum_v[...] = esum_v[...] + e * e
                    sumsq_v[...] = sumsq_v[...] + xw * xw
                    colsum_v[pl.ds(d0, 16)] = colsum_v[pl.ds(d0, 16)] + xw

        def wait_for(tl, rows_ref, sem):
            pltpu.make_async_copy(
                wdec_hbm.at[tidx_v.at[pl.ds(tl * K, K)]], rows_ref,
                sem).wait()

        gather(0, rows0_v, sem0)
        gather(1, rows1_v, sem1)

        def pair_body(p, c):
            @pl.when(lax.rem(p, jnp.int32(_XCH // 2)) == 0)
            def _():
                pltpu.sync_copy(
                    x_hbm.at[pl.ds((tbase + p * 2) * D_IN, _XCH * D_IN)],
                    x_v)
            wait_for(p * 2, rows0_v, sem0)
            compute(p * 2, rows0_v)

            @pl.when(p < tpw // 2 - 1)
            def _():
                gather(p * 2 + 2, rows0_v, sem0)
            wait_for(p * 2 + 1, rows1_v, sem1)
            compute(p * 2 + 1, rows1_v)

            @pl.when(p < tpw // 2 - 1)
            def _():
                gather(p * 2 + 3, rows1_v, sem1)

            @pl.when(lax.rem(p, jnp.int32(_XCH // 2)) == _XCH // 2 - 1)
            def _():
                pltpu.sync_copy(
                    sae_v,
                    sae_hbm.at[pl.ds((tbase + (p * 2 - _XCH + 2)) * D_IN,
                                     _XCH * D_IN)])
            return c
        lax.fori_loop(0, tpw // 2, pair_body, 0)

        pltpu.sync_copy(esum_v, esum_hbm.at[wid])
        pltpu.sync_copy(sumsq_v, sumsq_hbm.at[wid])
        pltpu.sync_copy(colsum_v, colsum_hbm.at[wid])

    f = functools.partial(
        pl.kernel,
        out_type=[
            jax.ShapeDtypeStruct((n * D_IN,), jnp.float32),
            jax.ShapeDtypeStruct((_NW, 16), jnp.float32),
            jax.ShapeDtypeStruct((_NW, D_IN), jnp.float32),
            jax.ShapeDtypeStruct((_NW, 16), jnp.float32),
        ],
        mesh=plsc.VectorSubcoreMesh(core_axis_name="c", subcore_axis_name="s"),
        compiler_params=pltpu.CompilerParams(
            use_tc_tiling_on_sc=False, needs_layout_passes=False),
        scratch_types=[
            pltpu.VMEM((tpw * K,), jnp.int32),
            pltpu.VMEM((tpw * K,), jnp.float32),
            pltpu.VMEM((D_IN,), jnp.float32),
            pltpu.VMEM((K, D_IN), jnp.float32),
            pltpu.VMEM((K, D_IN), jnp.float32),
            pltpu.VMEM((_XCH * D_IN,), jnp.float32),
            pltpu.VMEM((_XCH * D_IN,), jnp.float32),
            pltpu.VMEM((D_IN,), jnp.float32),
            pltpu.VMEM((16,), jnp.float32),
            pltpu.VMEM((16,), jnp.float32),
            pltpu.SemaphoreType.DMA,
            pltpu.SemaphoreType.DMA,
        ],
    )(body)
    return f(W_dec, tidx_flat, ta_flat, x_flat, b_dec)


def kernel(x, W_enc, b_enc, W_dec, b_dec):
    W_enc_bf = W_enc.astype(jnp.bfloat16)
    nh = N // _SPLIT
    parts = []
    for s in range(_SPLIT):
        xs = lax.slice_in_dim(x, s * nh, (s + 1) * nh, axis=0)
        acts, gmax = _encode(xs, W_enc_bf, b_enc, b_dec)
        grow3, cidx = _b1(gmax)
        cand_rows = _c_gather(grow3.reshape(nh * K),
                              acts.reshape(nh * NGRP, G))
        ta3, ti3 = _b2(cand_rows.reshape(nh, NCAND), cidx)
        sae_flat, esum, colsum, sumsq = _d_decode(
            W_dec, ti3.reshape(nh * K), ta3.reshape(nh * K),
            xs.reshape(nh * D_IN), b_dec)
        parts.append((sae_flat.reshape(nh, D_IN), ta3.reshape(nh, K),
                      ti3.reshape(nh, K), esum, colsum, sumsq))

    sae_out = jnp.concatenate([p[0] for p in parts], axis=0)
    top_acts = jnp.concatenate([p[1] for p in parts], axis=0)
    top_indices = jnp.concatenate([p[2] for p in parts], axis=0)
    esum_t = sum(jnp.sum(p[3]) for p in parts)
    colsum_t = sum(jnp.sum(p[4], axis=0) for p in parts)
    sumsq_t = sum(jnp.sum(p[5]) for p in parts)
    total_variance = sumsq_t - jnp.sum(colsum_t * colsum_t) / N
    fvu = esum_t / total_variance
    auxk_loss = jnp.zeros((), dtype=x.dtype)
    return sae_out, top_acts, top_indices, fvu, auxk_loss
